# Initial kernel scaffold; baseline (speedup 1.0000x reference)
import jax
import jax.numpy as jnp
from jax.experimental import pallas as pl


def kernel(src, dst, table, W1, b1, W2, b2):
    raise NotImplementedError("write your pallas kernel here")


# speedup vs baseline: 1.7851x; 1.7851x over previous
import functools

import jax
import jax.numpy as jnp
from jax import lax
from jax.experimental import pallas as pl
from jax.experimental.pallas import tpu as pltpu
from jax.experimental.pallas import tpu_sc as plsc

E = 320000
N = 10000
D = 128
H = 32
CO = 65

NC = 2
NS = 16
NW = NC * NS
EPW = E // NW

IDXW = 100
SUB = 2
CH = IDXW * SUB
NCHUNK = EPW // CH


def _gather_add_body(src_hbm, dst_hbm, table_hbm, out_hbm,
                     sidx, didx, abuf, bbuf, gsem):
    wid = lax.axis_index("s") * NC + lax.axis_index("c")
    irow0 = wid * (EPW // IDXW)
    ebase = wid * EPW

    def chunk(k, carry):
        irow = irow0 + k * SUB
        pltpu.sync_copy(src_hbm.at[pl.ds(irow, SUB)], sidx)
        pltpu.sync_copy(dst_hbm.at[pl.ds(irow, SUB)], didx)
        copies = []
        for j in range(SUB):
            copies.append(pltpu.async_copy(
                table_hbm.at[sidx.at[j]], abuf.at[pl.ds(j * IDXW, IDXW)], gsem))
            copies.append(pltpu.async_copy(
                table_hbm.at[didx.at[j]], bbuf.at[pl.ds(j * IDXW, IDXW)], gsem))
        for c in copies:
            c.wait()

        def add_row(r, carry2):
            for c in range(D // 16):
                sl = pl.ds(c * 16, 16)
                abuf[r, sl] = abuf[r, sl] + bbuf[r, sl]
            return carry2
        lax.fori_loop(0, CH, add_row, 0, unroll=2)

        pltpu.sync_copy(abuf, out_hbm.at[pl.ds(ebase + k * CH, CH)])
        return carry

    lax.fori_loop(0, NCHUNK, chunk, 0)


@functools.lru_cache(maxsize=None)
def _make_gather_add():
    return pl.kernel(
        _gather_add_body,
        out_type=jax.ShapeDtypeStruct((E, D), jnp.float32),
        mesh=plsc.VectorSubcoreMesh(core_axis_name="c", subcore_axis_name="s",
                                    num_cores=NC, num_subcores=NS),
        scratch_types=[
            pltpu.VMEM((SUB, IDXW), jnp.int32),
            pltpu.VMEM((SUB, IDXW), jnp.int32),
            pltpu.VMEM((CH, D), jnp.float32),
            pltpu.VMEM((CH, D), jnp.float32),
            pltpu.SemaphoreType.DMA,
        ],
    )


BLK = 2000


def _mlp_body(e_ref, w1_ref, b1_ref, w2_ref, b2_ref, o_ref):
    h = jnp.dot(e_ref[...], w1_ref[...], preferred_element_type=jnp.float32)
    h = jnp.maximum(h + b1_ref[...], 0.0)
    o = jnp.dot(h, w2_ref[...], preferred_element_type=jnp.float32)
    o = jnp.maximum(o + b2_ref[...], 0.0)
    m = jnp.max(o, axis=1, keepdims=True)
    p = jnp.exp(o - m)
    o_ref[...] = p / jnp.sum(p, axis=1, keepdims=True)


def _mlp(e, W1, b1, W2, b2):
    grid = (E // BLK,)
    return pl.pallas_call(
        _mlp_body,
        grid=grid,
        in_specs=[
            pl.BlockSpec((BLK, D), lambda i: (i, 0)),
            pl.BlockSpec((D, H), lambda i: (0, 0)),
            pl.BlockSpec((1, H), lambda i: (0, 0)),
            pl.BlockSpec((H, CO), lambda i: (0, 0)),
            pl.BlockSpec((1, CO), lambda i: (0, 0)),
        ],
        out_specs=pl.BlockSpec((BLK, CO), lambda i: (i, 0)),
        out_shape=jax.ShapeDtypeStruct((E, CO), jnp.float32),
    )(e, W1, b1, W2, b2)


def kernel(src, dst, table, W1, b1, W2, b2):
    src2d = src.reshape(E // IDXW, IDXW)
    dst2d = dst.reshape(E // IDXW, IDXW)
    e = _make_gather_add()(src2d, dst2d, table)
    return _mlp(e, W1, b1.reshape(1, H), W2, b2.reshape(1, CO))


# speedup vs baseline: 2.0119x; 1.1270x over previous
import functools

import jax
import jax.numpy as jnp
from jax import lax
from jax.experimental import pallas as pl
from jax.experimental.pallas import tpu as pltpu
from jax.experimental.pallas import tpu_sc as plsc

E = 320000
N = 10000
D = 128
H = 32
CO = 65

NC = 2
NS = 16
NW = NC * NS
EPW = E // NW

CH = 80
NROWS = EPW // CH
NBUF = 5
NT = NROWS // NBUF


def _gather_add_body(src_hbm, dst_hbm, table_hbm, out_hbm,
                     ibuf,
                     ab0, ab1, ab2, ab3, ab4, bb0, bb1, bb2, bb3, bb4,
                     g0, g1, g2, g3, g4, s0, s1, s2, s3, s4):
    abufs = (ab0, ab1, ab2, ab3, ab4)
    bbufs = (bb0, bb1, bb2, bb3, bb4)
    gsems = (g0, g1, g2, g3, g4)
    ssems = (s0, s1, s2, s3, s4)
    wid = lax.axis_index("s") * NC + lax.axis_index("c")
    ebase = wid * EPW

    def fetch_idx(t):
        pltpu.sync_copy(src_hbm.at[wid, t], ibuf.at[0])
        pltpu.sync_copy(dst_hbm.at[wid, t], ibuf.at[1])

    def issue(b):
        pltpu.async_copy(table_hbm.at[ibuf.at[0, b]], abufs[b], gsems[b])
        pltpu.async_copy(table_hbm.at[ibuf.at[1, b]], bbufs[b], gsems[b])

    def wait_gather(b):
        pltpu.make_async_copy(
            table_hbm.at[ibuf.at[0, b]], abufs[b], gsems[b]).wait()
        pltpu.make_async_copy(
            table_hbm.at[ibuf.at[1, b]], bbufs[b], gsems[b]).wait()

    def wait_store(cc, b):
        pltpu.make_async_copy(
            abufs[b], out_hbm.at[pl.ds(ebase + cc * CH, CH)], ssems[b]).wait()

    fetch_idx(0)
    for b in range(NBUF):
        issue(b)

    def outer(t, carry):
        c0 = t * NBUF
        for b in range(NBUF):
            cc = c0 + b
            wait_gather(b)
            a, bb = abufs[b], bbufs[b]

            def add_row(r, carry2, a=a, bb=bb):
                for q in range(D // 16):
                    sl = pl.ds(q * 16, 16)
                    a[r, sl] = a[r, sl] + bb[r, sl]
                return carry2
            lax.fori_loop(0, CH, add_row, 0, unroll=2)
            pltpu.async_copy(a, out_hbm.at[pl.ds(ebase + cc * CH, CH)],
                             ssems[b])

        @pl.when(t < NT - 1)
        def _prep():
            fetch_idx(t + 1)
            for b in range(NBUF):
                wait_store(c0 + b, b)
                issue(b)
        return carry

    lax.fori_loop(0, NT, outer, 0)
    for b in range(NBUF):
        wait_store(0, b)


@functools.lru_cache(maxsize=None)
def _make_gather_add():
    return pl.kernel(
        _gather_add_body,
        out_type=jax.ShapeDtypeStruct((E, D), jnp.float32),
        mesh=plsc.VectorSubcoreMesh(core_axis_name="c", subcore_axis_name="s",
                                    num_cores=NC, num_subcores=NS),
        scratch_types=[
            pltpu.VMEM((2, NBUF, CH), jnp.int32),
        ] + [pltpu.VMEM((CH, D), jnp.float32) for _ in range(2 * NBUF)]
          + [pltpu.SemaphoreType.DMA for _ in range(2 * NBUF)],
    )


BLK = 2000


def _mlp_body(e_ref, w1_ref, b1_ref, w2_ref, b2_ref, o_ref):
    h = jnp.dot(e_ref[...], w1_ref[...], preferred_element_type=jnp.float32)
    h = jnp.maximum(h + b1_ref[...], 0.0)
    o = jnp.dot(h, w2_ref[...], preferred_element_type=jnp.float32)
    o = jnp.maximum(o + b2_ref[...], 0.0)
    m = jnp.max(o, axis=1, keepdims=True)
    p = jnp.exp(o - m)
    o_ref[...] = p / jnp.sum(p, axis=1, keepdims=True)


def _mlp(e, W1, b1, W2, b2):
    grid = (E // BLK,)
    return pl.pallas_call(
        _mlp_body,
        grid=grid,
        in_specs=[
            pl.BlockSpec((BLK, D), lambda i: (i, 0)),
            pl.BlockSpec((D, H), lambda i: (0, 0)),
            pl.BlockSpec((1, H), lambda i: (0, 0)),
            pl.BlockSpec((H, CO), lambda i: (0, 0)),
            pl.BlockSpec((1, CO), lambda i: (0, 0)),
        ],
        out_specs=pl.BlockSpec((BLK, CO), lambda i: (i, 0)),
        out_shape=jax.ShapeDtypeStruct((E, CO), jnp.float32),
    )(e, W1, b1, W2, b2)


def kernel(src, dst, table, W1, b1, W2, b2):
    src2d = src.reshape(NW, NT, NBUF, CH)
    dst2d = dst.reshape(NW, NT, NBUF, CH)
    e = _make_gather_add()(src2d, dst2d, table)
    return _mlp(e, W1, b1.reshape(1, H), W2, b2.reshape(1, CO))
